# baseline (speedup 1.0000x reference)
import functools

import jax
import jax.numpy as jnp
from jax import lax
from jax.experimental import pallas as pl
from jax.experimental.pallas import tpu as pltpu
from jax.experimental.pallas import tpu_sc as plsc

B = 16
T_SRC = 512
MAX_MEL = 2048
D = 256
NUM_BINS = 256

def _dot(a, b):
    return jnp.dot(a, b, preferred_element_type=jnp.float32)


_N_WORKERS = 32
_HALF = MAX_MEL // 2
_GCHUNK = 128


def _regulate_body(src_hbm, dur_hbm, out_hbm, mel_hbm,
                   dur_v, cum_v, idx_v, mel_v, rows_v, rows2_v,
                   sem, sem2, wsem, wsem2):
    cid = lax.axis_index("c")
    sid = lax.axis_index("s")
    b = sid
    half = (sid + cid) % 2
    base = half * _HALF

    pltpu.sync_copy(dur_hbm.at[b], dur_v)

    def cum_step(i, carry):
        v = dur_v[pl.ds(i * 16, 16)]
        cs = plsc.cumsum(v) + carry
        cum_v[pl.ds(i * 16, 16)] = cs
        return carry + jnp.sum(v)

    lax.fori_loop(0, T_SRC // 16, cum_step, jnp.int32(0), unroll=False)

    def bs_step(j, _):
        pos = base + j * 16 + lax.iota(jnp.int32, 16)
        lo = jnp.zeros((16,), jnp.int32)
        hi = jnp.full((16,), T_SRC, jnp.int32)

        def halve(_, lh):
            lo, hi = lh
            mid = jnp.minimum((lo + hi) >> 1, T_SRC - 1)
            cv = plsc.load_gather(cum_v, [mid])
            le = cv <= pos
            return jnp.where(le, mid + 1, lo), jnp.where(le, hi, mid)

        lo, hi = lax.fori_loop(0, 10, halve, (lo, hi))
        idx_v[pl.ds(j * 16, 16)] = jnp.minimum(lo, T_SRC - 1) + b * T_SRC
        return 0

    def compute_idx_chunk(cc):
        lax.fori_loop(cc * (_GCHUNK // 16), (cc + 1) * (_GCHUNK // 16),
                      bs_step, 0, unroll=False)

    @pl.when(half == 0)
    def _():
        mel_v[...] = cum_v[pl.ds(T_SRC - 16, 16)]
        pltpu.sync_copy(mel_v, mel_hbm.at[b])

    n_chunks = _HALF // _GCHUNK
    bufs = (rows_v, rows2_v)
    gsems = (sem, sem2)
    wsems = (wsem, wsem2)

    def start_gather(cc):
        return pltpu.async_copy(
            src_hbm.at[idx_v.at[pl.ds(cc * _GCHUNK, _GCHUNK)]],
            bufs[cc % 2], gsems[cc % 2])

    def start_write(cc):
        return pltpu.async_copy(
            bufs[cc % 2],
            out_hbm.at[pl.ds(b * MAX_MEL + base + cc * _GCHUNK, _GCHUNK)],
            wsems[cc % 2])

    compute_idx_chunk(0)
    gathers = [start_gather(0)]
    compute_idx_chunk(1)
    gathers.append(start_gather(1))
    for cc in range(n_chunks):
        p = cc % 2
        gathers[cc].wait()
        wr = start_write(cc)
        if cc + 2 < n_chunks:
            compute_idx_chunk(cc + 2)
            wr.wait()
            gathers.append(start_gather(cc + 2))
        else:
            wr.wait()


@jax.jit
def _regulate(src_flat, dur):
    mesh = plsc.VectorSubcoreMesh(core_axis_name="c", subcore_axis_name="s")
    return pl.kernel(
        _regulate_body,
        mesh=mesh,
        compiler_params=pltpu.CompilerParams(needs_layout_passes=False),
        out_type=[
            jax.ShapeDtypeStruct((B * MAX_MEL, D), jnp.float32),
            jax.ShapeDtypeStruct((B, 16), jnp.int32),
        ],
        scratch_types=[
            pltpu.VMEM((T_SRC,), jnp.int32),
            pltpu.VMEM((T_SRC,), jnp.int32),
            pltpu.VMEM((_HALF,), jnp.int32),
            pltpu.VMEM((16,), jnp.int32),
            pltpu.VMEM((_GCHUNK, D), jnp.float32),
            pltpu.VMEM((_GCHUNK, D), jnp.float32),
            pltpu.SemaphoreType.DMA,
            pltpu.SemaphoreType.DMA,
            pltpu.SemaphoreType.DMA,
            pltpu.SemaphoreType.DMA,
        ],
    )(src_flat, dur)


def _conv3(x, w_ref, b_ref):
    h = _dot(x, w_ref[1]) + b_ref[0]
    a = _dot(x, w_ref[0])
    c = _dot(x, w_ref[2])
    z = jnp.zeros((1, h.shape[1]), jnp.float32)
    h = h + jnp.concatenate([z, a[:-1]], axis=0)
    h = h + jnp.concatenate([c[1:], z], axis=0)
    return h


def _layer_norm(h, g_ref, be_ref, ones_col):
    s1 = _dot(h, ones_col)
    s2 = _dot(h * h, ones_col)
    m = s1 * (1.0 / D)
    v = s2 * (1.0 / D) - m * m
    d = h - m
    return d * lax.rsqrt(v + 1e-5) * g_ref[0] + be_ref[0]


def _predictor(x, p_refs):
    w1, b1, g1, be1, w2, b2, g2, be2, wl, bl = p_refs
    ones_col = jnp.ones((D, 1), jnp.float32)
    h = jnp.maximum(_conv3(x, w1, b1), 0.0)
    h = _layer_norm(h, g1, be1, ones_col)
    h = jnp.maximum(_conv3(h, w2, b2), 0.0)
    h = _layer_norm(h, g2, be2, ones_col)
    return _dot(h, wl[...]) + bl[0]


def _bucket_embed(s, scale_ref, emb_ref):
    T = s.shape[0]
    step = (scale_ref[0][None, :] < s).astype(jnp.float32)
    ones = jnp.ones((T, 1), jnp.float32)
    zero = jnp.zeros((T, 1), jnp.float32)
    u = jnp.concatenate([ones, step[:, : NUM_BINS - 1]], axis=1)
    t = jnp.concatenate([step[:, : NUM_BINS - 1], zero], axis=1)
    return _dot(u - t, emb_ref[...])


def _pe_kernel_body(mel_ref, x_ref, src_ref, *refs):
    pp = refs[0:10]
    ep = refs[10:20]
    dp = refs[20:30]
    pscale, pemb, escale, eemb = refs[30:34]
    y_ref, p_ref, e_ref, ld_ref = refs[34:38]

    T = MAX_MEL
    mask = (lax.broadcasted_iota(jnp.int32, (T, 1), 0) < mel_ref[0])
    x = x_ref[0] * mask.astype(jnp.float32)
    ld_ref[0] = _predictor(src_ref[0], dp)
    p = _predictor(x, pp)
    x = x + _bucket_embed(p, pscale, pemb)
    e = _predictor(x, ep)
    y = x + _bucket_embed(e, escale, eemb)
    y_ref[0] = y
    p_ref[0] = p
    e_ref[0] = e


def _full_spec(a):
    nd = len(a.shape)
    return pl.BlockSpec(a.shape, lambda b, _n=nd: (0,) * _n)


def _prep_params(p):
    w1, b1, g1, be1, w2, b2, g2, be2, wl, bl = p
    return (w1, b1.reshape(1, D), g1.reshape(1, D), be1.reshape(1, D),
            w2, b2.reshape(1, D), g2.reshape(1, D), be2.reshape(1, D),
            wl.reshape(D, 1), bl.reshape(1, 1))


@jax.jit
def _pitch_energy(reg, mel16, src_seq, dur_params, pitch_params, energy_params,
                  pitch_emb, energy_emb, pitch_scale, energy_scale):
    pp = _prep_params(pitch_params)
    ep = _prep_params(energy_params)
    dp = _prep_params(dur_params)
    mel = mel16[:, 15].reshape(B, 1, 1)
    x = reg.reshape(B, MAX_MEL, D)
    extras = (pitch_scale.reshape(1, NUM_BINS).astype(jnp.float32), pitch_emb,
              energy_scale.reshape(1, NUM_BINS).astype(jnp.float32), energy_emb)
    y, p, e, ld = pl.pallas_call(
        _pe_kernel_body,
        grid=(B,),
        in_specs=[pl.BlockSpec((1, 1, 1), lambda b: (b, 0, 0)),
                  pl.BlockSpec((1, MAX_MEL, D), lambda b: (b, 0, 0)),
                  pl.BlockSpec((1, T_SRC, D), lambda b: (b, 0, 0))]
        + [_full_spec(a) for a in pp]
        + [_full_spec(a) for a in ep]
        + [_full_spec(a) for a in dp]
        + [_full_spec(a) for a in extras],
        out_specs=[pl.BlockSpec((1, MAX_MEL, D), lambda b: (b, 0, 0)),
                   pl.BlockSpec((1, MAX_MEL, 1), lambda b: (b, 0, 0)),
                   pl.BlockSpec((1, MAX_MEL, 1), lambda b: (b, 0, 0)),
                   pl.BlockSpec((1, T_SRC, 1), lambda b: (b, 0, 0))],
        out_shape=[jax.ShapeDtypeStruct((B, MAX_MEL, D), jnp.float32),
                   jax.ShapeDtypeStruct((B, MAX_MEL, 1), jnp.float32),
                   jax.ShapeDtypeStruct((B, MAX_MEL, 1), jnp.float32),
                   jax.ShapeDtypeStruct((B, T_SRC, 1), jnp.float32)],
    )(mel, x, src_seq, *pp, *ep, *dp, *extras)
    return y, p[:, :, 0], e[:, :, 0], ld[:, :, 0]


def kernel(src_seq, duration_target, max_mel_length, dur_params, pitch_params,
           energy_params, pitch_emb, energy_emb, pitch_scale, energy_scale):
    dur = duration_target.astype(jnp.int32)
    src_flat = src_seq.reshape(B * T_SRC, D)
    reg, mel16 = _regulate(src_flat, dur)
    output, pitch, energy, log_duration = _pitch_energy(
        reg, mel16, src_seq, dur_params, pitch_params, energy_params,
        pitch_emb, energy_emb, pitch_scale, energy_scale)
    mel_length = mel16[:, 15]
    return (output, log_duration, pitch, energy, mel_length)


# speedup vs baseline: 1.0810x; 1.0810x over previous
import functools

import jax
import jax.numpy as jnp
from jax import lax
from jax.experimental import pallas as pl
from jax.experimental.pallas import tpu as pltpu
from jax.experimental.pallas import tpu_sc as plsc

B = 16
T_SRC = 512
MAX_MEL = 2048
D = 256
NUM_BINS = 256

def _dot(a, b):
    return jnp.dot(a, b, preferred_element_type=jnp.float32)


_N_WORKERS = 32
_HALF = MAX_MEL // 2
_GCHUNK = 128


def _regulate_body(src_hbm, dur_hbm, out_hbm, mel_hbm,
                   dur_v, cum_v, idx_v, mel_v, rows_v, rows2_v, rows3_v,
                   sem, sem2, sem3, wsem, wsem2, wsem3):
    cid = lax.axis_index("c")
    sid = lax.axis_index("s")
    b = sid
    half = (sid + cid) % 2
    base = half * _HALF

    pltpu.sync_copy(dur_hbm.at[b], dur_v)

    def cum_step(i, carry):
        v = dur_v[pl.ds(i * 16, 16)]
        cs = plsc.cumsum(v) + carry
        cum_v[pl.ds(i * 16, 16)] = cs
        return carry + jnp.sum(v)

    lax.fori_loop(0, T_SRC // 16, cum_step, jnp.int32(0), unroll=False)

    def bs_step(j, _):
        pos = base + j * 16 + lax.iota(jnp.int32, 16)
        lo = jnp.zeros((16,), jnp.int32)
        hi = jnp.full((16,), T_SRC, jnp.int32)

        def halve(_, lh):
            lo, hi = lh
            mid = jnp.minimum((lo + hi) >> 1, T_SRC - 1)
            cv = plsc.load_gather(cum_v, [mid])
            le = cv <= pos
            return jnp.where(le, mid + 1, lo), jnp.where(le, hi, mid)

        lo, hi = lax.fori_loop(0, 10, halve, (lo, hi))
        idx_v[pl.ds(j * 16, 16)] = jnp.minimum(lo, T_SRC - 1) + b * T_SRC
        return 0

    def compute_idx_chunk(cc):
        lax.fori_loop(cc * (_GCHUNK // 16), (cc + 1) * (_GCHUNK // 16),
                      bs_step, 0, unroll=False)

    @pl.when(half == 0)
    def _():
        mel_v[...] = cum_v[pl.ds(T_SRC - 16, 16)]
        pltpu.sync_copy(mel_v, mel_hbm.at[b])

    n_chunks = _HALF // _GCHUNK
    bufs = (rows_v, rows2_v, rows3_v)
    gsems = (sem, sem2, sem3)
    wsems = (wsem, wsem2, wsem3)

    def start_gather(cc):
        return pltpu.async_copy(
            src_hbm.at[idx_v.at[pl.ds(cc * _GCHUNK, _GCHUNK)]],
            bufs[cc % 3], gsems[cc % 3])

    def start_write(cc):
        return pltpu.async_copy(
            bufs[cc % 3],
            out_hbm.at[pl.ds(b * MAX_MEL + base + cc * _GCHUNK, _GCHUNK)],
            wsems[cc % 3])

    compute_idx_chunk(0)
    gathers = [start_gather(0)]
    compute_idx_chunk(1)
    gathers.append(start_gather(1))
    writes = [None] * n_chunks
    for cc in range(n_chunks):
        gathers[cc].wait()
        writes[cc] = start_write(cc)
        q = cc + 2
        if q < n_chunks:
            compute_idx_chunk(q)
            if q - 3 >= 0:
                writes[q - 3].wait()
            gathers.append(start_gather(q))
    for cc in range(n_chunks - 3, n_chunks):
        writes[cc].wait()


@jax.jit
def _regulate(src_flat, dur):
    mesh = plsc.VectorSubcoreMesh(core_axis_name="c", subcore_axis_name="s")
    return pl.kernel(
        _regulate_body,
        mesh=mesh,
        compiler_params=pltpu.CompilerParams(needs_layout_passes=False),
        out_type=[
            jax.ShapeDtypeStruct((B * MAX_MEL, D), jnp.float32),
            jax.ShapeDtypeStruct((B, 16), jnp.int32),
        ],
        scratch_types=[
            pltpu.VMEM((T_SRC,), jnp.int32),
            pltpu.VMEM((T_SRC,), jnp.int32),
            pltpu.VMEM((_HALF,), jnp.int32),
            pltpu.VMEM((16,), jnp.int32),
            pltpu.VMEM((_GCHUNK, D), jnp.float32),
            pltpu.VMEM((_GCHUNK, D), jnp.float32),
            pltpu.VMEM((_GCHUNK, D), jnp.float32),
            pltpu.SemaphoreType.DMA,
            pltpu.SemaphoreType.DMA,
            pltpu.SemaphoreType.DMA,
            pltpu.SemaphoreType.DMA,
            pltpu.SemaphoreType.DMA,
            pltpu.SemaphoreType.DMA,
        ],
    )(src_flat, dur)


def _conv3(x, w_ref, b_ref):
    h = _dot(x, w_ref[1]) + b_ref[0]
    a = _dot(x, w_ref[0])
    c = _dot(x, w_ref[2])
    z = jnp.zeros((1, h.shape[1]), jnp.float32)
    h = h + jnp.concatenate([z, a[:-1]], axis=0)
    h = h + jnp.concatenate([c[1:], z], axis=0)
    return h


def _layer_norm(h, g_ref, be_ref, ones_col):
    s1 = _dot(h, ones_col)
    s2 = _dot(h * h, ones_col)
    m = s1 * (1.0 / D)
    v = s2 * (1.0 / D) - m * m
    d = h - m
    return d * lax.rsqrt(v + 1e-5) * g_ref[0] + be_ref[0]


def _predictor(x, p_refs):
    w1, b1, g1, be1, w2, b2, g2, be2, wl, bl = p_refs
    ones_col = jnp.ones((D, 1), jnp.float32)
    h = jnp.maximum(_conv3(x, w1, b1), 0.0)
    h = _layer_norm(h, g1, be1, ones_col)
    h = jnp.maximum(_conv3(h, w2, b2), 0.0)
    h = _layer_norm(h, g2, be2, ones_col)
    return _dot(h, wl[...]) + bl[0]


def _bucket_embed(s, scale_ref, emb_ref):
    T = s.shape[0]
    step = (scale_ref[0][None, :] < s).astype(jnp.float32)
    ones = jnp.ones((T, 1), jnp.float32)
    zero = jnp.zeros((T, 1), jnp.float32)
    u = jnp.concatenate([ones, step[:, : NUM_BINS - 1]], axis=1)
    t = jnp.concatenate([step[:, : NUM_BINS - 1], zero], axis=1)
    return _dot(u - t, emb_ref[...])


def _dur_kernel_body(x_ref, *refs):
    p_refs = refs[:10]
    out_ref = refs[10]
    out_ref[0] = _predictor(x_ref[0], p_refs)


def _pe_kernel_body(mel_ref, x_ref, *refs):
    pp = refs[0:10]
    ep = refs[10:20]
    pscale, pemb, escale, eemb = refs[20:24]
    y_ref, p_ref, e_ref = refs[24:27]

    T = MAX_MEL
    mask = (lax.broadcasted_iota(jnp.int32, (T, 1), 0) < mel_ref[0])
    x = x_ref[0] * mask.astype(jnp.float32)
    p = _predictor(x, pp)
    x = x + _bucket_embed(p, pscale, pemb)
    e = _predictor(x, ep)
    y = x + _bucket_embed(e, escale, eemb)
    y_ref[0] = y
    p_ref[0] = p
    e_ref[0] = e


def _full_spec(a):
    nd = len(a.shape)
    return pl.BlockSpec(a.shape, lambda b, _n=nd: (0,) * _n)


def _prep_params(p):
    w1, b1, g1, be1, w2, b2, g2, be2, wl, bl = p
    return (w1, b1.reshape(1, D), g1.reshape(1, D), be1.reshape(1, D),
            w2, b2.reshape(1, D), g2.reshape(1, D), be2.reshape(1, D),
            wl.reshape(D, 1), bl.reshape(1, 1))


@jax.jit
def _dur_predict(src_seq, dur_params):
    ps = _prep_params(dur_params)
    out = pl.pallas_call(
        _dur_kernel_body,
        grid=(B,),
        in_specs=[pl.BlockSpec((1, T_SRC, D), lambda b: (b, 0, 0))]
        + [_full_spec(a) for a in ps],
        out_specs=pl.BlockSpec((1, T_SRC, 1), lambda b: (b, 0, 0)),
        out_shape=jax.ShapeDtypeStruct((B, T_SRC, 1), jnp.float32),
    )(src_seq, *ps)
    return out[:, :, 0]


@jax.jit
def _pitch_energy(reg, mel16, pitch_params, energy_params,
                  pitch_emb, energy_emb, pitch_scale, energy_scale):
    pp = _prep_params(pitch_params)
    ep = _prep_params(energy_params)
    mel = mel16[:, 15].reshape(B, 1, 1)
    x = reg.reshape(B, MAX_MEL, D)
    extras = (pitch_scale.reshape(1, NUM_BINS).astype(jnp.float32), pitch_emb,
              energy_scale.reshape(1, NUM_BINS).astype(jnp.float32), energy_emb)
    y, p, e = pl.pallas_call(
        _pe_kernel_body,
        grid=(B,),
        in_specs=[pl.BlockSpec((1, 1, 1), lambda b: (b, 0, 0)),
                  pl.BlockSpec((1, MAX_MEL, D), lambda b: (b, 0, 0))]
        + [_full_spec(a) for a in pp]
        + [_full_spec(a) for a in ep]
        + [_full_spec(a) for a in extras],
        out_specs=[pl.BlockSpec((1, MAX_MEL, D), lambda b: (b, 0, 0)),
                   pl.BlockSpec((1, MAX_MEL, 1), lambda b: (b, 0, 0)),
                   pl.BlockSpec((1, MAX_MEL, 1), lambda b: (b, 0, 0))],
        out_shape=[jax.ShapeDtypeStruct((B, MAX_MEL, D), jnp.float32),
                   jax.ShapeDtypeStruct((B, MAX_MEL, 1), jnp.float32),
                   jax.ShapeDtypeStruct((B, MAX_MEL, 1), jnp.float32)],
    )(mel, x, *pp, *ep, *extras)
    return y, p[:, :, 0], e[:, :, 0]


def kernel(src_seq, duration_target, max_mel_length, dur_params, pitch_params,
           energy_params, pitch_emb, energy_emb, pitch_scale, energy_scale):
    dur = duration_target.astype(jnp.int32)
    src_flat = src_seq.reshape(B * T_SRC, D)
    reg, mel16 = _regulate(src_flat, dur)
    log_duration = _dur_predict(src_seq, dur_params)
    output, pitch, energy = _pitch_energy(
        reg, mel16, pitch_params, energy_params,
        pitch_emb, energy_emb, pitch_scale, energy_scale)
    mel_length = mel16[:, 15]
    return (output, log_duration, pitch, energy, mel_length)


# speedup vs baseline: 1.0818x; 1.0007x over previous
import functools

import jax
import jax.numpy as jnp
from jax import lax
from jax.experimental import pallas as pl
from jax.experimental.pallas import tpu as pltpu
from jax.experimental.pallas import tpu_sc as plsc

B = 16
T_SRC = 512
MAX_MEL = 2048
D = 256
NUM_BINS = 256

def _dot(a, b):
    return jnp.dot(a, b, preferred_element_type=jnp.float32)


_N_WORKERS = 32
_HALF = MAX_MEL // 2
_GCHUNK = 128


def _regulate_body(src_hbm, dur_hbm, out_hbm, mel_hbm,
                   dur_v, cum_v, idx_v, mel_v, rows_v, rows2_v, rows3_v,
                   sem, sem2, sem3, wsem, wsem2, wsem3):
    cid = lax.axis_index("c")
    sid = lax.axis_index("s")
    b = sid
    half = (sid + cid) % 2
    base = half * _HALF

    pltpu.sync_copy(dur_hbm.at[b], dur_v)

    def cum_step(i, carry):
        v = dur_v[pl.ds(i * 16, 16)]
        cs = plsc.cumsum(v) + carry
        cum_v[pl.ds(i * 16, 16)] = cs
        return carry + jnp.sum(v)

    lax.fori_loop(0, T_SRC // 16, cum_step, jnp.int32(0), unroll=False)

    def bs_step(j, _):
        pos = base + j * 16 + lax.iota(jnp.int32, 16)
        lo = jnp.zeros((16,), jnp.int32)
        hi = jnp.full((16,), T_SRC, jnp.int32)

        def halve(_, lh):
            lo, hi = lh
            mid = jnp.minimum((lo + hi) >> 1, T_SRC - 1)
            cv = plsc.load_gather(cum_v, [mid])
            le = cv <= pos
            return jnp.where(le, mid + 1, lo), jnp.where(le, hi, mid)

        lo, hi = lax.fori_loop(0, 10, halve, (lo, hi))
        idx_v[pl.ds(j * 16, 16)] = jnp.minimum(lo, T_SRC - 1) + b * T_SRC
        return 0

    def compute_idx_chunk(cc):
        lax.fori_loop(cc * (_GCHUNK // 16), (cc + 1) * (_GCHUNK // 16),
                      bs_step, 0, unroll=False)

    @pl.when(half == 0)
    def _():
        mel_v[...] = cum_v[pl.ds(T_SRC - 16, 16)]
        pltpu.sync_copy(mel_v, mel_hbm.at[b])

    n_chunks = _HALF // _GCHUNK
    bufs = (rows_v, rows2_v, rows3_v)
    gsems = (sem, sem2, sem3)
    wsems = (wsem, wsem2, wsem3)

    def start_gather(cc):
        return pltpu.async_copy(
            src_hbm.at[idx_v.at[pl.ds(cc * _GCHUNK, _GCHUNK)]],
            bufs[cc % 3], gsems[cc % 3])

    def start_write(cc):
        return pltpu.async_copy(
            bufs[cc % 3],
            out_hbm.at[pl.ds(b * MAX_MEL + base + cc * _GCHUNK, _GCHUNK)],
            wsems[cc % 3])

    compute_idx_chunk(0)
    gathers = [start_gather(0)]
    compute_idx_chunk(1)
    gathers.append(start_gather(1))
    writes = [None] * n_chunks
    for cc in range(n_chunks):
        gathers[cc].wait()
        writes[cc] = start_write(cc)
        q = cc + 2
        if q < n_chunks:
            compute_idx_chunk(q)
            if q - 3 >= 0:
                writes[q - 3].wait()
            gathers.append(start_gather(q))
    for cc in range(n_chunks - 3, n_chunks):
        writes[cc].wait()


@jax.jit
def _regulate(src_flat, dur):
    mesh = plsc.VectorSubcoreMesh(core_axis_name="c", subcore_axis_name="s")
    return pl.kernel(
        _regulate_body,
        mesh=mesh,
        compiler_params=pltpu.CompilerParams(needs_layout_passes=False),
        out_type=[
            jax.ShapeDtypeStruct((B * MAX_MEL, D), jnp.float32),
            jax.ShapeDtypeStruct((B, 16), jnp.int32),
        ],
        scratch_types=[
            pltpu.VMEM((T_SRC,), jnp.int32),
            pltpu.VMEM((T_SRC,), jnp.int32),
            pltpu.VMEM((_HALF,), jnp.int32),
            pltpu.VMEM((16,), jnp.int32),
            pltpu.VMEM((_GCHUNK, D), jnp.float32),
            pltpu.VMEM((_GCHUNK, D), jnp.float32),
            pltpu.VMEM((_GCHUNK, D), jnp.float32),
            pltpu.SemaphoreType.DMA,
            pltpu.SemaphoreType.DMA,
            pltpu.SemaphoreType.DMA,
            pltpu.SemaphoreType.DMA,
            pltpu.SemaphoreType.DMA,
            pltpu.SemaphoreType.DMA,
        ],
    )(src_flat, dur)


def _conv3(x, w_ref, b_ref):
    h = _dot(x, w_ref[1]) + b_ref[0]
    a = _dot(x, w_ref[0])
    c = _dot(x, w_ref[2])
    z = jnp.zeros((1, h.shape[1]), jnp.float32)
    h = h + jnp.concatenate([z, a[:-1]], axis=0)
    h = h + jnp.concatenate([c[1:], z], axis=0)
    return h


def _layer_norm(h, g_ref, be_ref, ones_col):
    s1 = _dot(h, ones_col)
    s2 = _dot(h * h, ones_col)
    m = s1 * (1.0 / D)
    v = s2 * (1.0 / D) - m * m
    d = h - m
    return d * lax.rsqrt(v + 1e-5) * g_ref[0] + be_ref[0]


def _predictor(x, p_refs):
    w1, b1, g1, be1, w2, b2, g2, be2, wl, bl = p_refs
    ones_col = jnp.ones((D, 1), jnp.float32)
    h = jnp.maximum(_conv3(x, w1, b1), 0.0)
    h = _layer_norm(h, g1, be1, ones_col)
    h = jnp.maximum(_conv3(h, w2, b2), 0.0)
    h = _layer_norm(h, g2, be2, ones_col)
    return _dot(h, wl[...]) + bl[0]


def _bucket_embed(s, scale_ref, emb_ref):
    T = s.shape[0]
    step = (scale_ref[0][None, :] < s).astype(jnp.float32)
    ones = jnp.ones((T, 1), jnp.float32)
    zero = jnp.zeros((T, 1), jnp.float32)
    u = jnp.concatenate([ones, step[:, : NUM_BINS - 1]], axis=1)
    t = jnp.concatenate([step[:, : NUM_BINS - 1], zero], axis=1)
    return _dot(u - t, emb_ref[...])


def _dur_kernel_body(x_ref, *refs):
    p_refs = refs[:10]
    out_ref = refs[10]
    out_ref[0] = _predictor(x_ref[0], p_refs)


def _pe_kernel_body(mel_ref, x_ref, *refs):
    pp = refs[0:10]
    ep = refs[10:20]
    pscale, pemb, escale, eemb = refs[20:24]
    y_ref, p_ref, e_ref = refs[24:27]

    T = MAX_MEL
    mask = (lax.broadcasted_iota(jnp.int32, (T, 1), 0) < mel_ref[0])
    x = x_ref[0] * mask.astype(jnp.float32)
    p = _predictor(x, pp)
    x = x + _bucket_embed(p, pscale, pemb)
    e = _predictor(x, ep)
    y = x + _bucket_embed(e, escale, eemb)
    y_ref[0] = y
    p_ref[0] = p
    e_ref[0] = e


def _full_spec(a):
    nd = len(a.shape)
    return pl.BlockSpec(a.shape, lambda b, _n=nd: (0,) * _n)


def _prep_params(p):
    w1, b1, g1, be1, w2, b2, g2, be2, wl, bl = p
    return (w1, b1.reshape(1, D), g1.reshape(1, D), be1.reshape(1, D),
            w2, b2.reshape(1, D), g2.reshape(1, D), be2.reshape(1, D),
            wl.reshape(D, 1), bl.reshape(1, 1))


@jax.jit
def _dur_predict(src_seq, dur_params):
    ps = _prep_params(dur_params)
    out = pl.pallas_call(
        _dur_kernel_body,
        grid=(B,),
        in_specs=[pl.BlockSpec((1, T_SRC, D), lambda b: (b, 0, 0))]
        + [_full_spec(a) for a in ps],
        out_specs=pl.BlockSpec((1, T_SRC, 1), lambda b: (b, 0, 0)),
        out_shape=jax.ShapeDtypeStruct((B, T_SRC, 1), jnp.float32),
    )(src_seq, *ps)
    return out[:, :, 0]


@jax.jit
def _pitch_energy(reg, mel16, pitch_params, energy_params,
                  pitch_emb, energy_emb, pitch_scale, energy_scale):
    pp = _prep_params(pitch_params)
    ep = _prep_params(energy_params)
    mel = mel16[:, 15].reshape(B, 1, 1)
    x = reg.reshape(B, MAX_MEL, D)
    extras = (pitch_scale.reshape(1, NUM_BINS).astype(jnp.float32), pitch_emb,
              energy_scale.reshape(1, NUM_BINS).astype(jnp.float32), energy_emb)
    y, p, e = pl.pallas_call(
        _pe_kernel_body,
        grid=(B,),
        in_specs=[pl.BlockSpec((1, 1, 1), lambda b: (b, 0, 0)),
                  pl.BlockSpec((1, MAX_MEL, D), lambda b: (b, 0, 0))]
        + [_full_spec(a) for a in pp]
        + [_full_spec(a) for a in ep]
        + [_full_spec(a) for a in extras],
        out_specs=[pl.BlockSpec((1, MAX_MEL, D), lambda b: (b, 0, 0)),
                   pl.BlockSpec((1, MAX_MEL, 1), lambda b: (b, 0, 0)),
                   pl.BlockSpec((1, MAX_MEL, 1), lambda b: (b, 0, 0))],
        out_shape=[jax.ShapeDtypeStruct((B, MAX_MEL, D), jnp.float32),
                   jax.ShapeDtypeStruct((B, MAX_MEL, 1), jnp.float32),
                   jax.ShapeDtypeStruct((B, MAX_MEL, 1), jnp.float32)],
    )(mel, x, *pp, *ep, *extras)
    return y, p[:, :, 0], e[:, :, 0]


def kernel(src_seq, duration_target, max_mel_length, dur_params, pitch_params,
           energy_params, pitch_emb, energy_emb, pitch_scale, energy_scale):
    dur = duration_target.astype(jnp.int32)
    src_flat = src_seq.reshape(B * T_SRC, D)
    log_duration = _dur_predict(src_seq, dur_params)
    reg, mel16 = _regulate(src_flat, dur)
    output, pitch, energy = _pitch_energy(
        reg, mel16, pitch_params, energy_params,
        pitch_emb, energy_emb, pitch_scale, energy_scale)
    mel_length = mel16[:, 15]
    return (output, log_duration, pitch, energy, mel_length)
